# Initial kernel scaffold; baseline (speedup 1.0000x reference)
import jax
import jax.numpy as jnp
from jax.experimental import pallas as pl


def kernel(x, edge_index, edge_weight, W1, b1, W2, b2):
    raise NotImplementedError("write your pallas kernel here")


# speedup vs baseline: 4.0402x; 4.0402x over previous
import functools

import jax
import jax.numpy as jnp
from jax import lax
from jax.experimental import pallas as pl
from jax.experimental.pallas import tpu as pltpu
from jax.experimental.pallas import tpu_sc as plsc

NC = 2
NS = 16
L = 16

CHUNK = 80


def _sc_propagate(x, src, dst, w):
    n, d = x.shape
    e = src.shape[0]
    nw = NC * NS
    e_per_w = e // nw
    n_chunks = e_per_w // CHUNK
    npad = ((n + 8 * NS - 1) // (8 * NS)) * (8 * NS)
    rows_per_tile = npad // NS
    zrows = 128

    mesh = plsc.VectorSubcoreMesh(
        core_axis_name="c", subcore_axis_name="s", num_cores=NC, num_subcores=NS
    )

    @functools.partial(
        pl.kernel,
        out_type=jax.ShapeDtypeStruct((NC * npad, d), jnp.float32),
        mesh=mesh,
        scratch_types=[
            pltpu.VMEM((CHUNK,), jnp.int32),
            pltpu.VMEM((CHUNK,), jnp.int32),
            pltpu.VMEM((CHUNK,), jnp.float32),
            pltpu.VMEM((CHUNK, d), jnp.float32),
            pltpu.VMEM((zrows, d), jnp.float32),
            pltpu.VMEM_SHARED((npad, d), jnp.float32),
            pltpu.SemaphoreType.DMA,
        ],
        compiler_params=pltpu.CompilerParams(needs_layout_passes=False),
    )
    def k(x_hbm, src_hbm, dst_hbm, w_hbm, out_hbm, sidx, didx, wv, rows, zbuf, acc, sem):
        c = lax.axis_index("c")
        s = lax.axis_index("s")
        wid = c * NS + s

        def zrow(r, _):
            for kk in range(d // L):
                zbuf[r, pl.ds(kk * L, L)] = jnp.zeros((L,), jnp.float32)
            return 0

        lax.fori_loop(0, zrows, zrow, 0)
        row0 = s * rows_per_tile
        for j in range(rows_per_tile // zrows):
            pltpu.sync_copy(zbuf, acc.at[pl.ds(row0 + j * zrows, zrows)])
        plsc.subcore_barrier()

        def chunk_body(i, _):
            base = pl.multiple_of(wid * e_per_w + i * CHUNK, 8)
            pltpu.sync_copy(src_hbm.at[pl.ds(base, CHUNK)], sidx)
            pltpu.sync_copy(dst_hbm.at[pl.ds(base, CHUNK)], didx)
            pltpu.sync_copy(w_hbm.at[pl.ds(base, CHUNK)], wv)
            pltpu.async_copy(x_hbm.at[sidx], rows, sem).wait()

            def edge_body(ei, _):
                wsplat = plsc.load_gather(wv, [jnp.full((L,), ei, jnp.int32)])
                for kk in range(d // L):
                    sl = pl.ds(kk * L, L)
                    rows[ei, sl] = rows[ei, sl] * wsplat
                return 0

            lax.fori_loop(0, CHUNK, edge_body, 0)
            pltpu.sync_copy(rows, acc.at[didx], add=True)
            return 0

        lax.fori_loop(0, n_chunks, chunk_body, 0)
        plsc.subcore_barrier()

        pltpu.sync_copy(
            acc.at[pl.ds(row0, rows_per_tile)],
            out_hbm.at[pl.ds(c * npad + row0, rows_per_tile)],
        )

    return k(x, src, dst, w)


def _tc_dense(p0, p1, x, w1, b1, w2, b2):
    n, d = x.shape
    bm = 2000

    def body(p0_ref, p1_ref, x_ref, w1_ref, b1_ref, w2_ref, b2_ref, o_ref):
        xp = p0_ref[...] + p1_ref[...]
        h1 = xp + x_ref[...]
        h2 = xp * x_ref[...]
        dn = (((1,), (1,)), ((), ()))
        acc = lax.dot_general(h1, w1_ref[...], dn, preferred_element_type=jnp.float32)
        acc = acc + lax.dot_general(h2, w2_ref[...], dn, preferred_element_type=jnp.float32)
        acc = acc + b1_ref[...] + b2_ref[...]
        o_ref[...] = jnp.maximum(acc, 0.0)

    row_spec = pl.BlockSpec((bm, d), lambda i: (i, 0))
    full_spec = pl.BlockSpec((d, d), lambda i: (0, 0))
    bias_spec = pl.BlockSpec((1, d), lambda i: (0, 0))
    return pl.pallas_call(
        body,
        out_shape=jax.ShapeDtypeStruct((n, d), jnp.float32),
        grid=(n // bm,),
        in_specs=[row_spec, row_spec, row_spec, full_spec, bias_spec, full_spec, bias_spec],
        out_specs=row_spec,
    )(p0, p1, x, w1, b1.reshape(1, d), w2, b2.reshape(1, d))


def kernel(x, edge_index, edge_weight, W1, b1, W2, b2):
    n, d = x.shape
    src = edge_index[0].astype(jnp.int32)
    dst = edge_index[1].astype(jnp.int32)
    pflat = _sc_propagate(x, src, dst, edge_weight.astype(jnp.float32))
    npad = pflat.shape[0] // NC
    return _tc_dense(pflat[:n], pflat[npad:npad + n], x, W1, b1, W2, b2)
